# baseline (speedup 1.0000x reference)
import functools

import jax
import jax.numpy as jnp
from jax import lax
from jax.experimental import pallas as pl
from jax.experimental.pallas import tpu as pltpu
from jax.experimental.pallas import tpu_sc as plsc

B_DIM, L_DIM = 16384, 50
N_TOKENS = B_DIM * L_DIM
N_PAIRS = N_TOKENS // 2
EMBED = 64
HALF_B = B_DIM // 2

NC, NS = 2, 16
NW = NC * NS
NSEG = 2
SEG_L = L_DIM // NSEG
SEG_TOK = N_TOKENS // NSEG
ROWS_PER_W = SEG_TOK // NW
CHUNK = 800
NCHUNK = ROWS_PER_W // CHUNK


def _sc_gather(ids_flat, table):
    mesh = plsc.VectorSubcoreMesh(core_axis_name="c", subcore_axis_name="s")

    @functools.partial(
        pl.kernel,
        out_type=jax.ShapeDtypeStruct((SEG_TOK, EMBED), jnp.float32),
        mesh=mesh,
        compiler_params=pltpu.CompilerParams(use_tc_tiling_on_sc=False),
        scratch_types=[
            pltpu.VMEM((ROWS_PER_W,), jnp.int32),
            pltpu.VMEM((2, CHUNK, EMBED), jnp.float32),
            pltpu.SemaphoreType.DMA,
            pltpu.SemaphoreType.DMA,
            pltpu.SemaphoreType.DMA,
        ],
    )
    def gather_kernel(ids_hbm, table_hbm, out_hbm, idx_v, rows_v, sg0, sg1, sw):
        wid = lax.axis_index("s") * NC + lax.axis_index("c")
        base = wid * ROWS_PER_W
        pltpu.sync_copy(ids_hbm.at[pl.ds(pl.multiple_of(base, ROWS_PER_W), ROWS_PER_W)], idx_v)
        sems = (sg0, sg1)

        def gather_start(c, buf):
            pltpu.async_copy(
                table_hbm.at[idx_v.at[pl.ds(c * CHUNK, CHUNK)]],
                rows_v.at[buf], sems[buf])

        def gather_drain(buf):
            pltpu.make_async_copy(
                table_hbm.at[pl.ds(0, CHUNK)], rows_v.at[buf], sems[buf]).wait()

        def writeback_start(c, buf):
            cb = pl.multiple_of(base + c * CHUNK, CHUNK)
            pltpu.async_copy(rows_v.at[buf], out_hbm.at[pl.ds(cb, CHUNK)], sw)

        def writeback_drain(buf):
            pltpu.make_async_copy(
                table_hbm.at[pl.ds(0, CHUNK)], rows_v.at[buf], sw).wait()

        gather_start(0, 0)

        def body(c2, carry):
            for b in (0, 1):
                c = c2 * 2 + b
                gather_drain(b)

                @pl.when(c + 1 < NCHUNK)
                def _():
                    @pl.when(c >= 1)
                    def _():
                        writeback_drain(1 - b)
                    gather_start(c + 1, 1 - b)

                writeback_start(c, b)
            return carry

        lax.fori_loop(0, NCHUNK // 2, body, 0, unroll=False)
        writeback_drain(0)
        writeback_drain(1)

    return gather_kernel(ids_flat, table)


def _tc_combine_kernel(ce_ref, co_ref, g_ref, w_ref, ctbl_ref, out_ref):
    _tc_combine_body(ce_ref, co_ref, g_ref, w_ref, ctbl_ref, out_ref)


def _tc_combine_kernel_acc(ce_ref, co_ref, g_ref, w_ref, ctbl_ref, prev_ref,
                           out_ref):
    del prev_ref
    _tc_combine_body(ce_ref, co_ref, g_ref, w_ref, ctbl_ref, out_ref)


def _tc_combine_body(ce_ref, co_ref, g_ref, w_ref, ctbl_ref, out_ref):
    c_e = ce_ref[0, 0, :]
    c_o = co_ref[0, 0, :]
    iota = lax.broadcasted_iota(jnp.int32, (32, HALF_B), 0)
    sel = jnp.where((iota & 1) == 0, c_e[None, :], c_o[None, :])
    oh = ((iota >> 1) == sel).astype(jnp.float32)
    bias = lax.dot_general(ctbl_ref[...], oh,
                           (((1,), (0,)), ((), ())),
                           preferred_element_type=jnp.float32)
    res = lax.dot_general(w_ref[...], g_ref[...].astype(jnp.bfloat16),
                          (((1,), (1,)), ((), ())),
                          preferred_element_type=jnp.float32)
    t = jnp.tanh(res + bias)
    out_ref[0, :, 0:HALF_B] = t[0:EMBED, :]
    out_ref[0, :, HALF_B:B_DIM] = t[EMBED:2 * EMBED, :]


def _tc_combine(c_e, c_o, gathered, weo, ctbl, seg, out_prev):
    base_specs = [
        pl.BlockSpec((1, 1, HALF_B), lambda l: (l, 0, 0)),
        pl.BlockSpec((1, 1, HALF_B), lambda l: (l, 0, 0)),
        pl.BlockSpec((HALF_B, 2 * EMBED), lambda l: (l, 0)),
        pl.BlockSpec((2 * EMBED, 2 * EMBED), lambda l: (0, 0)),
        pl.BlockSpec((2 * EMBED, 32), lambda l: (0, 0)),
    ]
    off = seg * SEG_L
    out_spec = pl.BlockSpec((1, EMBED, B_DIM), lambda l: (l + off, 0, 0))
    out_shape = jax.ShapeDtypeStruct((L_DIM, EMBED, B_DIM), jnp.float32)
    if seg == 0:
        return pl.pallas_call(
            _tc_combine_kernel,
            grid=(SEG_L,),
            in_specs=base_specs,
            out_specs=out_spec,
            out_shape=out_shape,
        )(c_e, c_o, gathered, weo, ctbl)
    return pl.pallas_call(
        _tc_combine_kernel_acc,
        grid=(SEG_L,),
        in_specs=base_specs + [pl.BlockSpec(memory_space=pltpu.MemorySpace.HBM)],
        out_specs=out_spec,
        out_shape=out_shape,
        input_output_aliases={5: 0},
    )(c_e, c_o, gathered, weo, ctbl, out_prev)


def kernel(task_ids, difficulty, task_type, task_table, diff_table, type_table, W, b):
    tid3 = task_ids.T.astype(jnp.int32).reshape(L_DIM, 2, HALF_B)
    ids_perm = tid3.transpose(0, 2, 1).reshape(-1)

    dbias = diff_table @ W[:, EMBED:EMBED + 8].T
    tbias = type_table @ W[:, EMBED + 8:EMBED + 16].T
    ctbl9 = (dbias[:, None, :] + tbias[None, :, :] + b).reshape(9, EMBED)
    c4 = jnp.zeros((2, EMBED, 16, 2), jnp.float32)
    c4 = c4.at[0, :, :9, 0].set(ctbl9.T).at[1, :, :9, 1].set(ctbl9.T)
    ctbl2t = c4.reshape(2 * EMBED, 32)

    wt = W[:, :EMBED]
    zero = jnp.zeros((EMBED, EMBED), jnp.float32)
    weo = jnp.block([[wt, zero], [zero, wt]]).astype(jnp.bfloat16)

    combo_t = (difficulty.astype(jnp.int32) * 3
               + task_type.astype(jnp.int32)).T
    c_e = combo_t[:, :HALF_B].reshape(L_DIM, 1, HALF_B)
    c_o = combo_t[:, HALF_B:].reshape(L_DIM, 1, HALF_B)

    out_t = None
    for seg in range(NSEG):
        ids_seg = lax.slice(ids_perm, (seg * SEG_TOK,), ((seg + 1) * SEG_TOK,))
        g = _sc_gather(ids_seg, task_table)
        g = g.reshape(SEG_TOK // 2, 2 * EMBED)
        ls = slice(seg * SEG_L, (seg + 1) * SEG_L)
        out_t = _tc_combine(c_e[ls], c_o[ls], g, weo, ctbl2t, seg, out_t)
    return out_t.transpose(2, 0, 1)


# speedup vs baseline: 1.1178x; 1.1178x over previous
import functools

import jax
import jax.numpy as jnp
from jax import lax
from jax.experimental import pallas as pl
from jax.experimental.pallas import tpu as pltpu
from jax.experimental.pallas import tpu_sc as plsc

B_DIM, L_DIM = 16384, 50
N_TOKENS = B_DIM * L_DIM
N_PAIRS = N_TOKENS // 2
EMBED = 64
HALF_B = B_DIM // 2

NC, NS = 2, 16
NW = NC * NS
ROWS_PER_W = N_TOKENS // NW
CHUNK = 800
NCHUNK = ROWS_PER_W // CHUNK


def _sc_gather(ids_flat, table):
    mesh = plsc.VectorSubcoreMesh(core_axis_name="c", subcore_axis_name="s")

    @functools.partial(
        pl.kernel,
        out_type=jax.ShapeDtypeStruct((N_TOKENS, EMBED), jnp.float32),
        mesh=mesh,
        compiler_params=pltpu.CompilerParams(use_tc_tiling_on_sc=False),
        scratch_types=[
            pltpu.VMEM((ROWS_PER_W,), jnp.int32),
            pltpu.VMEM((2, CHUNK, EMBED), jnp.float32),
            pltpu.SemaphoreType.DMA,
            pltpu.SemaphoreType.DMA,
            pltpu.SemaphoreType.DMA,
        ],
    )
    def gather_kernel(ids_hbm, table_hbm, out_hbm, idx_v, rows_v, sg0, sg1, sw):
        wid = lax.axis_index("s") * NC + lax.axis_index("c")
        base = wid * ROWS_PER_W
        pltpu.sync_copy(ids_hbm.at[pl.ds(pl.multiple_of(base, ROWS_PER_W), ROWS_PER_W)], idx_v)
        sems = (sg0, sg1)

        def gather_start(c, buf):
            pltpu.async_copy(
                table_hbm.at[idx_v.at[pl.ds(c * CHUNK, CHUNK)]],
                rows_v.at[buf], sems[buf])

        def gather_drain(buf):
            pltpu.make_async_copy(
                table_hbm.at[pl.ds(0, CHUNK)], rows_v.at[buf], sems[buf]).wait()

        def writeback_start(c, buf):
            cb = pl.multiple_of(base + c * CHUNK, CHUNK)
            pltpu.async_copy(rows_v.at[buf], out_hbm.at[pl.ds(cb, CHUNK)], sw)

        def writeback_drain(buf):
            pltpu.make_async_copy(
                table_hbm.at[pl.ds(0, CHUNK)], rows_v.at[buf], sw).wait()

        gather_start(0, 0)

        def body(c2, carry):
            for b in (0, 1):
                c = c2 * 2 + b
                gather_drain(b)

                @pl.when(c + 1 < NCHUNK)
                def _():
                    @pl.when(c >= 1)
                    def _():
                        writeback_drain(1 - b)
                    gather_start(c + 1, 1 - b)

                writeback_start(c, b)
            return carry

        lax.fori_loop(0, NCHUNK // 2, body, 0, unroll=False)
        writeback_drain(0)
        writeback_drain(1)

    return gather_kernel(ids_flat, table)


def _tc_combine_kernel(ce_ref, co_ref, g_ref, w_ref, ctbl_ref, out_ref):
    c_e = ce_ref[0, 0, :]
    c_o = co_ref[0, 0, :]
    iota = lax.broadcasted_iota(jnp.int32, (32, HALF_B), 0)
    sel = jnp.where((iota & 1) == 0, c_e[None, :], c_o[None, :])
    oh = ((iota >> 1) == sel).astype(jnp.float32)
    bias = lax.dot_general(ctbl_ref[...], oh,
                           (((1,), (0,)), ((), ())),
                           preferred_element_type=jnp.float32)
    res = lax.dot_general(w_ref[...], g_ref[...].astype(jnp.bfloat16),
                          (((1,), (1,)), ((), ())),
                          preferred_element_type=jnp.float32)
    t = jnp.tanh(res + bias)
    out_ref[0, :, 0:HALF_B] = t[0:EMBED, :]
    out_ref[0, :, HALF_B:B_DIM] = t[EMBED:2 * EMBED, :]


def _tc_combine(c_e, c_o, gathered, weo, ctbl):
    return pl.pallas_call(
        _tc_combine_kernel,
        grid=(L_DIM,),
        in_specs=[
            pl.BlockSpec((1, 1, HALF_B), lambda l: (l, 0, 0)),
            pl.BlockSpec((1, 1, HALF_B), lambda l: (l, 0, 0)),
            pl.BlockSpec((HALF_B, 2 * EMBED), lambda l: (l, 0)),
            pl.BlockSpec((2 * EMBED, 2 * EMBED), lambda l: (0, 0)),
            pl.BlockSpec((2 * EMBED, 32), lambda l: (0, 0)),
        ],
        out_specs=pl.BlockSpec((1, EMBED, B_DIM), lambda l: (l, 0, 0)),
        out_shape=jax.ShapeDtypeStruct((L_DIM, EMBED, B_DIM), jnp.float32),
    )(c_e, c_o, gathered, weo, ctbl)


def kernel(task_ids, difficulty, task_type, task_table, diff_table, type_table, W, b):
    tid3 = task_ids.T.astype(jnp.int32).reshape(L_DIM, 2, HALF_B)
    ids_perm = tid3.transpose(0, 2, 1).reshape(-1)

    dbias = diff_table @ W[:, EMBED:EMBED + 8].T
    tbias = type_table @ W[:, EMBED + 8:EMBED + 16].T
    ctbl9 = (dbias[:, None, :] + tbias[None, :, :] + b).reshape(9, EMBED)
    c4 = jnp.zeros((2, EMBED, 16, 2), jnp.float32)
    c4 = c4.at[0, :, :9, 0].set(ctbl9.T).at[1, :, :9, 1].set(ctbl9.T)
    ctbl2t = c4.reshape(2 * EMBED, 32)

    wt = W[:, :EMBED]
    zero = jnp.zeros((EMBED, EMBED), jnp.float32)
    weo = jnp.block([[wt, zero], [zero, wt]]).astype(jnp.bfloat16)

    combo_t = (difficulty.astype(jnp.int32) * 3
               + task_type.astype(jnp.int32)).T
    c_e = combo_t[:, :HALF_B].reshape(L_DIM, 1, HALF_B)
    c_o = combo_t[:, HALF_B:].reshape(L_DIM, 1, HALF_B)

    gathered = _sc_gather(ids_perm, task_table)
    gathered = gathered.reshape(N_PAIRS, 2 * EMBED)
    out_t = _tc_combine(c_e, c_o, gathered, weo, ctbl2t)
    return out_t.transpose(2, 0, 1)
